# initial kernel scaffold (unmeasured)
import jax
import jax.numpy as jnp
from jax import lax
from jax.experimental import pallas as pl
from jax.experimental.pallas import tpu as pltpu

B, HS, WS, C = 2, 256, 256, 128
N_GLOBAL = 512 * 512
EPS = 1e-5
BLK = 16
NBLK = HS // BLK


def kernel(x, k, Wp):
    mx = lax.axis_index("x")
    my = lax.axis_index("y")

    r_edge = jnp.where(mx == 0, HS - 1, 0)
    c_edge = jnp.where(my == 0, WS - 1, 0)
    row_send = lax.dynamic_slice_in_dim(x, r_edge, 1, axis=1).reshape(B, WS, C)
    col_send = lax.dynamic_slice_in_dim(x, c_edge, 1, axis=2).reshape(B, HS, C)
    corner_send = lax.dynamic_slice(
        x, (0, r_edge, c_edge, 0), (B, 1, 1, C)
    ).reshape(B, C)

    def body(x_ref, k_ref, wp_ref, row_s_ref, col_s_ref, cor_s_ref,
             out_ref,
             hpad, slabA, slabB, outA, outB,
             halo_row, halo_col, halo_cor,
             stats_tx, stats_rx,
             in_sems, out_sems, halo_ssem, halo_rsem, st_ssem, st_rsem):
        mx = lax.axis_index("x")
        my = lax.axis_index("y")

        rdma_row = pltpu.make_async_remote_copy(
            src_ref=row_s_ref, dst_ref=halo_row,
            send_sem=halo_ssem.at[0], recv_sem=halo_rsem.at[0],
            device_id=(1 - mx, my), device_id_type=pl.DeviceIdType.MESH)
        rdma_col = pltpu.make_async_remote_copy(
            src_ref=col_s_ref, dst_ref=halo_col,
            send_sem=halo_ssem.at[1], recv_sem=halo_rsem.at[1],
            device_id=(mx, 1 - my), device_id_type=pl.DeviceIdType.MESH)
        rdma_cor = pltpu.make_async_remote_copy(
            src_ref=cor_s_ref, dst_ref=halo_cor,
            send_sem=halo_ssem.at[2], recv_sem=halo_rsem.at[2],
            device_id=(1 - mx, 1 - my), device_id_type=pl.DeviceIdType.MESH)
        rdma_row.start()
        rdma_col.start()
        rdma_cor.start()

        slabs = [slabA, slabB]

        def in_copy(b, slab, sem):
            return pltpu.make_async_copy(
                x_ref.at[:, pl.ds(b * BLK, BLK)], slab, sem)

        in_copy(0, slabs[0], in_sems.at[0]).start()
        s_acc = jnp.zeros((B, C), jnp.float32)
        q_acc = jnp.zeros((B, C), jnp.float32)
        for b in range(NBLK):
            sl = slabs[b % 2]
            in_copy(b, sl, in_sems.at[b % 2]).wait()
            if b + 1 < NBLK:
                in_copy(b + 1, slabs[(b + 1) % 2],
                        in_sems.at[(b + 1) % 2]).start()
            v = sl[...]
            s_acc = s_acc + jnp.sum(v, axis=(1, 2))
            q_acc = q_acc + jnp.sum(v * v, axis=(1, 2))
            hpad[:, 1 + b * BLK:1 + (b + 1) * BLK, 1:1 + WS, :] = (
                v.astype(jnp.bfloat16))

        stats_tx[0:B, :] = s_acc
        stats_tx[B:2 * B, :] = q_acc
        ex0 = pltpu.make_async_remote_copy(
            src_ref=stats_tx, dst_ref=stats_rx.at[0],
            send_sem=st_ssem.at[0], recv_sem=st_rsem.at[0],
            device_id=(1 - mx, my), device_id_type=pl.DeviceIdType.MESH)
        ex0.start()
        ex0.wait()
        comb = stats_tx[...] + stats_rx[0]
        stats_tx[...] = comb
        ex1 = pltpu.make_async_remote_copy(
            src_ref=stats_tx, dst_ref=stats_rx.at[1],
            send_sem=st_ssem.at[1], recv_sem=st_rsem.at[1],
            device_id=(mx, 1 - my), device_id_type=pl.DeviceIdType.MESH)
        ex1.start()
        ex1.wait()
        tot = comb + stats_rx[1]

        mean = tot[0:B, :] / N_GLOBAL
        var = tot[B:2 * B, :] / N_GLOBAL - mean * mean
        sigma = jnp.sqrt(var + EPS)
        rstd = 1.0 / sigma
        mean4 = mean.reshape(B, 1, 1, C)
        rstd4 = rstd.reshape(B, 1, 1, C)
        sigma4 = sigma.reshape(B, 1, 1, C)

        for b in range(NBLK):
            sl_ = hpad[:, 1 + b * BLK:1 + (b + 1) * BLK, 1:1 + WS, :]
            hpad[:, 1 + b * BLK:1 + (b + 1) * BLK, 1:1 + WS, :] = (
                (sl_.astype(jnp.float32) - mean4) * rstd4
            ).astype(jnp.bfloat16)

        rdma_row.wait()
        rdma_col.wait()
        rdma_cor.wait()
        hr = ((halo_row[...] - mean.reshape(B, 1, C))
              * rstd.reshape(B, 1, C)).astype(jnp.bfloat16)
        hc = ((halo_col[...] - mean.reshape(B, 1, C))
              * rstd.reshape(B, 1, C)).astype(jnp.bfloat16)
        hco = ((halo_cor[...] - mean) * rstd).astype(jnp.bfloat16)

        row_top = jnp.where(mx == 1, hr, hpad[:, 1, 1:1 + WS, :])
        row_bot = jnp.where(mx == 0, hr, hpad[:, HS, 1:1 + WS, :])
        hpad[:, 0, 1:1 + WS, :] = row_top
        hpad[:, HS + 1, 1:1 + WS, :] = row_bot
        col_lft = jnp.where(my == 1, hc, hpad[:, 1:1 + HS, 1, :])
        col_rgt = jnp.where(my == 0, hc, hpad[:, 1:1 + HS, WS, :])
        hpad[:, 1:1 + HS, 0, :] = col_lft
        hpad[:, 1:1 + HS, WS + 1, :] = col_rgt

        def corner(r_remote, c_remote, hr_end, hc_end, local):
            return jnp.where(
                r_remote,
                jnp.where(c_remote, hco, hr_end),
                jnp.where(c_remote, hc_end, local))

        hpad[:, 0, 0, :] = corner(
            mx == 1, my == 1, hr[:, 0, :], hc[:, 0, :], hpad[:, 1, 1, :])
        hpad[:, 0, WS + 1, :] = corner(
            mx == 1, my == 0, hr[:, WS - 1, :], hc[:, 0, :],
            hpad[:, 1, WS, :])
        hpad[:, HS + 1, 0, :] = corner(
            mx == 0, my == 1, hr[:, 0, :], hc[:, HS - 1, :],
            hpad[:, HS, 1, :])
        hpad[:, HS + 1, WS + 1, :] = corner(
            mx == 0, my == 0, hr[:, WS - 1, :], hc[:, HS - 1, :],
            hpad[:, HS, WS, :])

        kv = k_ref[...].astype(jnp.bfloat16)
        wv = wp_ref[...].astype(jnp.bfloat16)
        outs = [outA, outB]
        out_copies = [None, None]
        for b in range(NBLK):
            ob = outs[b % 2]
            if out_copies[b % 2] is not None:
                out_copies[b % 2].wait()
            r0 = b * BLK
            acc = None
            for di in range(3):
                for dj in range(3):
                    t = (hpad[:, r0 + di:r0 + di + BLK, dj:dj + WS, :]
                         * kv[di, dj, :])
                    acc = t if acc is None else acc + t
            cf = acc.astype(jnp.float32)
            a = (cf / (1.0 + jnp.exp(-cf))).astype(jnp.bfloat16)
            proj = jnp.dot(a.reshape(B * BLK * WS, C), wv,
                           preferred_element_type=jnp.float32
                           ).reshape(B, BLK, WS, C)
            h_ctr = hpad[:, 1 + r0:1 + r0 + BLK, 1:1 + WS, :]
            res = mean4 + h_ctr.astype(jnp.float32) * sigma4 + proj
            ob[...] = res.astype(jnp.bfloat16)
            cp = pltpu.make_async_copy(
                ob, out_ref.at[:, pl.ds(r0, BLK)], out_sems.at[b % 2])
            cp.start()
            out_copies[b % 2] = cp
        for cp in out_copies:
            if cp is not None:
                cp.wait()

    out_shape = jax.ShapeDtypeStruct((B, HS, WS, C), jnp.bfloat16)
    return pl.pallas_call(
        body,
        out_shape=out_shape,
        in_specs=[
            pl.BlockSpec(memory_space=pltpu.MemorySpace.ANY),
            pl.BlockSpec(memory_space=pltpu.MemorySpace.VMEM),
            pl.BlockSpec(memory_space=pltpu.MemorySpace.VMEM),
            pl.BlockSpec(memory_space=pltpu.MemorySpace.VMEM),
            pl.BlockSpec(memory_space=pltpu.MemorySpace.VMEM),
            pl.BlockSpec(memory_space=pltpu.MemorySpace.VMEM),
        ],
        out_specs=pl.BlockSpec(memory_space=pltpu.MemorySpace.ANY),
        scratch_shapes=[
            pltpu.VMEM((B, HS + 2, WS + 2, C), jnp.bfloat16),
            pltpu.VMEM((B, BLK, WS, C), jnp.float32),
            pltpu.VMEM((B, BLK, WS, C), jnp.float32),
            pltpu.VMEM((B, BLK, WS, C), jnp.bfloat16),
            pltpu.VMEM((B, BLK, WS, C), jnp.bfloat16),
            pltpu.VMEM((B, WS, C), jnp.float32),
            pltpu.VMEM((B, HS, C), jnp.float32),
            pltpu.VMEM((B, C), jnp.float32),
            pltpu.VMEM((2 * B, C), jnp.float32),
            pltpu.VMEM((2, 2 * B, C), jnp.float32),
            pltpu.SemaphoreType.DMA((2,)),
            pltpu.SemaphoreType.DMA((2,)),
            pltpu.SemaphoreType.DMA((3,)),
            pltpu.SemaphoreType.DMA((3,)),
            pltpu.SemaphoreType.DMA((2,)),
            pltpu.SemaphoreType.DMA((2,)),
        ],
        compiler_params=pltpu.CompilerParams(
            vmem_limit_bytes=128 * 1024 * 1024,
        ),
    )(x, k, Wp, row_send, col_send, corner_send)


# baseline (device time: 289598 ns/iter reference)
import jax
import jax.numpy as jnp
from jax import lax
from jax.experimental import pallas as pl
from jax.experimental.pallas import tpu as pltpu

B, HS, WS, C = 2, 256, 256, 128
N_GLOBAL = 512 * 512
EPS = 1e-5
BLK = 16
NBLK = HS // BLK


def kernel(x, k, Wp):
    mx = lax.axis_index("x")
    my = lax.axis_index("y")

    r_edge = jnp.where(mx == 0, HS - 1, 0)
    c_edge = jnp.where(my == 0, WS - 1, 0)
    row_send = lax.dynamic_slice_in_dim(x, r_edge, 1, axis=1).reshape(B, WS, C)
    col_send = lax.dynamic_slice_in_dim(x, c_edge, 1, axis=2).reshape(B, HS, C)
    corner_send = lax.dynamic_slice(
        x, (0, r_edge, c_edge, 0), (B, 1, 1, C)
    ).reshape(B, C)

    def body(x_ref, k_ref, wp_ref, row_s_ref, col_s_ref, cor_s_ref,
             out_ref,
             hpad, slab, outb,
             halo_row, halo_col, halo_cor,
             stats_tx, stats_rx,
             in_sem, out_sem, halo_ssem, halo_rsem, st_ssem, st_rsem):
        mx = lax.axis_index("x")
        my = lax.axis_index("y")

        rdma_row = pltpu.make_async_remote_copy(
            src_ref=row_s_ref, dst_ref=halo_row,
            send_sem=halo_ssem.at[0], recv_sem=halo_rsem.at[0],
            device_id=(1 - mx, my), device_id_type=pl.DeviceIdType.MESH)
        rdma_col = pltpu.make_async_remote_copy(
            src_ref=col_s_ref, dst_ref=halo_col,
            send_sem=halo_ssem.at[1], recv_sem=halo_rsem.at[1],
            device_id=(mx, 1 - my), device_id_type=pl.DeviceIdType.MESH)
        rdma_cor = pltpu.make_async_remote_copy(
            src_ref=cor_s_ref, dst_ref=halo_cor,
            send_sem=halo_ssem.at[2], recv_sem=halo_rsem.at[2],
            device_id=(1 - mx, 1 - my), device_id_type=pl.DeviceIdType.MESH)
        rdma_row.start()
        rdma_col.start()
        rdma_cor.start()

        def stepA(b, carry):
            s, q = carry
            cp = pltpu.make_async_copy(
                x_ref.at[:, pl.ds(b * BLK, BLK)], slab, in_sem)
            cp.start()
            cp.wait()
            v = slab[...]
            s = s + jnp.sum(v, axis=(1, 2))
            q = q + jnp.sum(v * v, axis=(1, 2))
            hpad[:, pl.ds(1 + b * BLK, BLK), 1:1 + WS, :] = (
                v.astype(jnp.bfloat16))
            return s, q

        s_acc, q_acc = lax.fori_loop(
            0, NBLK, stepA,
            (jnp.zeros((B, C), jnp.float32), jnp.zeros((B, C), jnp.float32)))

        stats_tx[0:B, :] = s_acc
        stats_tx[B:2 * B, :] = q_acc
        ex0 = pltpu.make_async_remote_copy(
            src_ref=stats_tx, dst_ref=stats_rx.at[0],
            send_sem=st_ssem.at[0], recv_sem=st_rsem.at[0],
            device_id=(1 - mx, my), device_id_type=pl.DeviceIdType.MESH)
        ex0.start()
        ex0.wait()
        comb = stats_tx[...] + stats_rx[0]
        stats_tx[...] = comb
        ex1 = pltpu.make_async_remote_copy(
            src_ref=stats_tx, dst_ref=stats_rx.at[1],
            send_sem=st_ssem.at[1], recv_sem=st_rsem.at[1],
            device_id=(mx, 1 - my), device_id_type=pl.DeviceIdType.MESH)
        ex1.start()
        ex1.wait()
        tot = comb + stats_rx[1]

        mean = tot[0:B, :] / N_GLOBAL
        var = tot[B:2 * B, :] / N_GLOBAL - mean * mean
        rstd = 1.0 / jnp.sqrt(var + EPS)

        rdma_row.wait()
        rdma_col.wait()
        rdma_cor.wait()
        hr = halo_row[...].astype(jnp.bfloat16)
        hc = halo_col[...].astype(jnp.bfloat16)
        hco = halo_cor[...].astype(jnp.bfloat16)

        row_top = jnp.where(mx == 1, hr, hpad[:, 1, 1:1 + WS, :])
        row_bot = jnp.where(mx == 0, hr, hpad[:, HS, 1:1 + WS, :])
        hpad[:, 0, 1:1 + WS, :] = row_top
        hpad[:, HS + 1, 1:1 + WS, :] = row_bot
        col_lft = jnp.where(my == 1, hc, hpad[:, 1:1 + HS, 1, :])
        col_rgt = jnp.where(my == 0, hc, hpad[:, 1:1 + HS, WS, :])
        hpad[:, 1:1 + HS, 0, :] = col_lft
        hpad[:, 1:1 + HS, WS + 1, :] = col_rgt

        def corner(r_remote, c_remote, hr_end, hc_end, local):
            return jnp.where(
                r_remote,
                jnp.where(c_remote, hco, hr_end),
                jnp.where(c_remote, hc_end, local))

        hpad[:, 0, 0, :] = corner(
            mx == 1, my == 1, hr[:, 0, :], hc[:, 0, :], hpad[:, 1, 1, :])
        hpad[:, 0, WS + 1, :] = corner(
            mx == 1, my == 0, hr[:, WS - 1, :], hc[:, 0, :],
            hpad[:, 1, WS, :])
        hpad[:, HS + 1, 0, :] = corner(
            mx == 0, my == 1, hr[:, 0, :], hc[:, HS - 1, :],
            hpad[:, HS, 1, :])
        hpad[:, HS + 1, WS + 1, :] = corner(
            mx == 0, my == 0, hr[:, WS - 1, :], hc[:, HS - 1, :],
            hpad[:, HS, WS, :])

        kv = k_ref[...].astype(jnp.bfloat16)
        wv = wp_ref[...].astype(jnp.bfloat16)
        ksum = jnp.sum(k_ref[...], axis=(0, 1))
        kkm4 = (mean * ksum).reshape(B, 1, 1, C)
        rstd4 = rstd.reshape(B, 1, 1, C)

        def stepC(b, _):
            r0 = b * BLK
            acc = jnp.zeros((B, BLK, WS, C), jnp.bfloat16)
            for di in range(3):
                for dj in range(3):
                    acc = acc + (hpad[:, pl.ds(r0 + di, BLK), dj:dj + WS, :]
                                 * kv[di, dj, :])
            cf = (acc.astype(jnp.float32) - kkm4) * rstd4
            a = (cf / (1.0 + jnp.exp(-cf))).astype(jnp.bfloat16)
            proj = jnp.dot(a.reshape(B * BLK * WS, C), wv,
                           preferred_element_type=jnp.float32
                           ).reshape(B, BLK, WS, C)
            res = (hpad[:, pl.ds(1 + r0, BLK), 1:1 + WS, :]
                   .astype(jnp.float32) + proj)
            outb[...] = res.astype(jnp.bfloat16)
            cp = pltpu.make_async_copy(
                outb, out_ref.at[:, pl.ds(r0, BLK)], out_sem)
            cp.start()
            cp.wait()
            return 0

        lax.fori_loop(0, NBLK, stepC, 0)

    out_shape = jax.ShapeDtypeStruct((B, HS, WS, C), jnp.bfloat16)
    return pl.pallas_call(
        body,
        out_shape=out_shape,
        in_specs=[
            pl.BlockSpec(memory_space=pl.ANY),
            pl.BlockSpec(memory_space=pltpu.MemorySpace.VMEM),
            pl.BlockSpec(memory_space=pltpu.MemorySpace.VMEM),
            pl.BlockSpec(memory_space=pltpu.MemorySpace.VMEM),
            pl.BlockSpec(memory_space=pltpu.MemorySpace.VMEM),
            pl.BlockSpec(memory_space=pltpu.MemorySpace.VMEM),
        ],
        out_specs=pl.BlockSpec(memory_space=pl.ANY),
        scratch_shapes=[
            pltpu.VMEM((B, HS + 2, WS + 2, C), jnp.bfloat16),
            pltpu.VMEM((B, BLK, WS, C), jnp.float32),
            pltpu.VMEM((B, BLK, WS, C), jnp.bfloat16),
            pltpu.VMEM((B, WS, C), jnp.float32),
            pltpu.VMEM((B, HS, C), jnp.float32),
            pltpu.VMEM((B, C), jnp.float32),
            pltpu.VMEM((2 * B, C), jnp.float32),
            pltpu.VMEM((2, 2 * B, C), jnp.float32),
            pltpu.SemaphoreType.DMA,
            pltpu.SemaphoreType.DMA,
            pltpu.SemaphoreType.DMA((3,)),
            pltpu.SemaphoreType.DMA((3,)),
            pltpu.SemaphoreType.DMA((2,)),
            pltpu.SemaphoreType.DMA((2,)),
        ],
        compiler_params=pltpu.CompilerParams(
            vmem_limit_bytes=128 * 1024 * 1024,
        ),
    )(x, k, Wp, row_send, col_send, corner_send)


# device time: 240453 ns/iter; 1.2044x vs baseline; 1.2044x over previous
import jax
import jax.numpy as jnp
from jax import lax
from jax.experimental import pallas as pl
from jax.experimental.pallas import tpu as pltpu

B, HS, WS, C = 2, 256, 256, 128
N_GLOBAL = 512 * 512
EPS = 1e-5
BLK = 16
NBLK = HS // BLK


def kernel(x, k, Wp):
    mx = lax.axis_index("x")
    my = lax.axis_index("y")

    r_edge = jnp.where(mx == 0, HS - 1, 0)
    c_edge = jnp.where(my == 0, WS - 1, 0)
    row_send = lax.dynamic_slice_in_dim(x, r_edge, 1, axis=1).reshape(B, WS, C)
    col_send = lax.dynamic_slice_in_dim(x, c_edge, 1, axis=2).reshape(B, HS, C)
    corner_send = lax.dynamic_slice(
        x, (0, r_edge, c_edge, 0), (B, 1, 1, C)
    ).reshape(B, C)

    def body(x_ref, k_ref, wp_ref, row_s_ref, col_s_ref, cor_s_ref,
             out_ref,
             hpad, slab, outb,
             halo_row, halo_col, halo_cor,
             stats_tx, stats_rx,
             in_sems, out_sems, halo_ssem, halo_rsem, st_ssem, st_rsem):
        mx = lax.axis_index("x")
        my = lax.axis_index("y")

        rdma_row = pltpu.make_async_remote_copy(
            src_ref=row_s_ref, dst_ref=halo_row,
            send_sem=halo_ssem.at[0], recv_sem=halo_rsem.at[0],
            device_id=(1 - mx, my), device_id_type=pl.DeviceIdType.MESH)
        rdma_col = pltpu.make_async_remote_copy(
            src_ref=col_s_ref, dst_ref=halo_col,
            send_sem=halo_ssem.at[1], recv_sem=halo_rsem.at[1],
            device_id=(mx, 1 - my), device_id_type=pl.DeviceIdType.MESH)
        rdma_cor = pltpu.make_async_remote_copy(
            src_ref=cor_s_ref, dst_ref=halo_cor,
            send_sem=halo_ssem.at[2], recv_sem=halo_rsem.at[2],
            device_id=(1 - mx, 1 - my), device_id_type=pl.DeviceIdType.MESH)
        rdma_row.start()
        rdma_col.start()
        rdma_cor.start()

        def in_copy(b, p):
            return pltpu.make_async_copy(
                x_ref.at[:, pl.ds(b * BLK, BLK)], slab.at[p], in_sems.at[p])

        in_copy(0, 0).start()

        def stepA(b, carry):
            s, q = carry
            p = lax.rem(b, 2)

            @pl.when(b + 1 < NBLK)
            def _():
                in_copy(b + 1, lax.rem(b + 1, 2)).start()

            in_copy(b, p).wait()
            v = slab[p]
            s = s + jnp.sum(v, axis=(1, 2))
            q = q + jnp.sum(v * v, axis=(1, 2))
            hpad[:, pl.ds(1 + b * BLK, BLK), 1:1 + WS, :] = (
                v.astype(jnp.bfloat16))
            return s, q

        s_acc, q_acc = lax.fori_loop(
            0, NBLK, stepA,
            (jnp.zeros((B, C), jnp.float32), jnp.zeros((B, C), jnp.float32)))

        stats_tx[0:B, :] = s_acc
        stats_tx[B:2 * B, :] = q_acc
        ex0 = pltpu.make_async_remote_copy(
            src_ref=stats_tx, dst_ref=stats_rx.at[0],
            send_sem=st_ssem.at[0], recv_sem=st_rsem.at[0],
            device_id=(1 - mx, my), device_id_type=pl.DeviceIdType.MESH)
        ex0.start()
        ex0.wait()
        comb = stats_tx[...] + stats_rx[0]
        stats_tx[...] = comb
        ex1 = pltpu.make_async_remote_copy(
            src_ref=stats_tx, dst_ref=stats_rx.at[1],
            send_sem=st_ssem.at[1], recv_sem=st_rsem.at[1],
            device_id=(mx, 1 - my), device_id_type=pl.DeviceIdType.MESH)
        ex1.start()
        ex1.wait()
        tot = comb + stats_rx[1]

        mean = tot[0:B, :] / N_GLOBAL
        var = tot[B:2 * B, :] / N_GLOBAL - mean * mean
        rstd = 1.0 / jnp.sqrt(var + EPS)

        rdma_row.wait()
        rdma_col.wait()
        rdma_cor.wait()
        hr = halo_row[...].astype(jnp.bfloat16)
        hc = halo_col[...].astype(jnp.bfloat16)
        hco = halo_cor[...].astype(jnp.bfloat16)

        row_top = jnp.where(mx == 1, hr, hpad[:, 1, 1:1 + WS, :])
        row_bot = jnp.where(mx == 0, hr, hpad[:, HS, 1:1 + WS, :])
        hpad[:, 0, 1:1 + WS, :] = row_top
        hpad[:, HS + 1, 1:1 + WS, :] = row_bot
        col_lft = jnp.where(my == 1, hc, hpad[:, 1:1 + HS, 1, :])
        col_rgt = jnp.where(my == 0, hc, hpad[:, 1:1 + HS, WS, :])
        hpad[:, 1:1 + HS, 0, :] = col_lft
        hpad[:, 1:1 + HS, WS + 1, :] = col_rgt

        def corner(r_remote, c_remote, hr_end, hc_end, local):
            return jnp.where(
                r_remote,
                jnp.where(c_remote, hco, hr_end),
                jnp.where(c_remote, hc_end, local))

        hpad[:, 0, 0, :] = corner(
            mx == 1, my == 1, hr[:, 0, :], hc[:, 0, :], hpad[:, 1, 1, :])
        hpad[:, 0, WS + 1, :] = corner(
            mx == 1, my == 0, hr[:, WS - 1, :], hc[:, 0, :],
            hpad[:, 1, WS, :])
        hpad[:, HS + 1, 0, :] = corner(
            mx == 0, my == 1, hr[:, 0, :], hc[:, HS - 1, :],
            hpad[:, HS, 1, :])
        hpad[:, HS + 1, WS + 1, :] = corner(
            mx == 0, my == 0, hr[:, WS - 1, :], hc[:, HS - 1, :],
            hpad[:, HS, WS, :])

        kv = k_ref[...].astype(jnp.bfloat16)
        wv = wp_ref[...].astype(jnp.bfloat16)
        ksum = jnp.sum(k_ref[...], axis=(0, 1))
        kkm4 = (mean * ksum).reshape(B, 1, 1, C)
        rstd4 = rstd.reshape(B, 1, 1, C)

        def out_copy(b, p):
            return pltpu.make_async_copy(
                outb.at[p], out_ref.at[:, pl.ds(b * BLK, BLK)],
                out_sems.at[p])

        def stepC(b, _):
            r0 = b * BLK
            p = lax.rem(b, 2)

            @pl.when(b >= 2)
            def _():
                out_copy(b - 2, p).wait()

            acc = jnp.zeros((B, BLK, WS, C), jnp.bfloat16)
            for di in range(3):
                for dj in range(3):
                    acc = acc + (hpad[:, pl.ds(r0 + di, BLK), dj:dj + WS, :]
                                 * kv[di, dj, :])
            cf = (acc.astype(jnp.float32) - kkm4) * rstd4
            a = (cf / (1.0 + jnp.exp(-cf))).astype(jnp.bfloat16)
            proj = jnp.dot(a.reshape(B * BLK * WS, C), wv,
                           preferred_element_type=jnp.float32
                           ).reshape(B, BLK, WS, C)
            res = (hpad[:, pl.ds(1 + r0, BLK), 1:1 + WS, :]
                   .astype(jnp.float32) + proj)
            outb[p] = res.astype(jnp.bfloat16)
            out_copy(b, p).start()
            return 0

        lax.fori_loop(0, NBLK, stepC, 0)
        out_copy(NBLK - 2, 0).wait()
        out_copy(NBLK - 1, 1).wait()

    out_shape = jax.ShapeDtypeStruct((B, HS, WS, C), jnp.bfloat16)
    return pl.pallas_call(
        body,
        out_shape=out_shape,
        in_specs=[
            pl.BlockSpec(memory_space=pl.ANY),
            pl.BlockSpec(memory_space=pltpu.MemorySpace.VMEM),
            pl.BlockSpec(memory_space=pltpu.MemorySpace.VMEM),
            pl.BlockSpec(memory_space=pltpu.MemorySpace.VMEM),
            pl.BlockSpec(memory_space=pltpu.MemorySpace.VMEM),
            pl.BlockSpec(memory_space=pltpu.MemorySpace.VMEM),
        ],
        out_specs=pl.BlockSpec(memory_space=pl.ANY),
        scratch_shapes=[
            pltpu.VMEM((B, HS + 2, WS + 2, C), jnp.bfloat16),
            pltpu.VMEM((2, B, BLK, WS, C), jnp.float32),
            pltpu.VMEM((2, B, BLK, WS, C), jnp.bfloat16),
            pltpu.VMEM((B, WS, C), jnp.float32),
            pltpu.VMEM((B, HS, C), jnp.float32),
            pltpu.VMEM((B, C), jnp.float32),
            pltpu.VMEM((2 * B, C), jnp.float32),
            pltpu.VMEM((2, 2 * B, C), jnp.float32),
            pltpu.SemaphoreType.DMA((2,)),
            pltpu.SemaphoreType.DMA((2,)),
            pltpu.SemaphoreType.DMA((3,)),
            pltpu.SemaphoreType.DMA((3,)),
            pltpu.SemaphoreType.DMA((2,)),
            pltpu.SemaphoreType.DMA((2,)),
        ],
        compiler_params=pltpu.CompilerParams(
            vmem_limit_bytes=128 * 1024 * 1024,
        ),
    )(x, k, Wp, row_send, col_send, corner_send)


# device time: 188598 ns/iter; 1.5355x vs baseline; 1.2749x over previous
import os

import jax
import jax.numpy as jnp
from jax import lax
from jax.experimental import pallas as pl
from jax.experimental.pallas import tpu as pltpu

B, HS, WS, C = 2, 256, 256, 128
N_GLOBAL = 512 * 512
EPS = 1e-5
BLK = 32
NBLK = HS // BLK


def kernel(x, k, Wp):
    mx = lax.axis_index("x")
    my = lax.axis_index("y")

    r_edge = jnp.where(mx == 0, HS - 1, 0)
    c_edge = jnp.where(my == 0, WS - 1, 0)
    row_send = lax.dynamic_slice_in_dim(x, r_edge, 1, axis=1).reshape(B, WS, C)
    col_send = lax.dynamic_slice_in_dim(x, c_edge, 1, axis=2).reshape(B, HS, C)
    corner_send = lax.dynamic_slice(
        x, (0, r_edge, c_edge, 0), (B, 1, 1, C)
    ).reshape(B, C)

    def body(x_ref, k_ref, wp_ref, row_s_ref, col_s_ref, cor_s_ref,
             out_ref,
             hpad, slab, outb,
             halo_row, halo_col, halo_cor,
             stats_tx, stats_rx,
             in_sems, out_sems, halo_ssem, halo_rsem, st_ssem, st_rsem):
        mx = lax.axis_index("x")
        my = lax.axis_index("y")

        rdma_row = pltpu.make_async_remote_copy(
            src_ref=row_s_ref, dst_ref=halo_row,
            send_sem=halo_ssem.at[0], recv_sem=halo_rsem.at[0],
            device_id=(1 - mx, my), device_id_type=pl.DeviceIdType.MESH)
        rdma_col = pltpu.make_async_remote_copy(
            src_ref=col_s_ref, dst_ref=halo_col,
            send_sem=halo_ssem.at[1], recv_sem=halo_rsem.at[1],
            device_id=(mx, 1 - my), device_id_type=pl.DeviceIdType.MESH)
        rdma_cor = pltpu.make_async_remote_copy(
            src_ref=cor_s_ref, dst_ref=halo_cor,
            send_sem=halo_ssem.at[2], recv_sem=halo_rsem.at[2],
            device_id=(1 - mx, 1 - my), device_id_type=pl.DeviceIdType.MESH)
        rdma_row.start()
        rdma_col.start()
        rdma_cor.start()

        def in_copy(b, p):
            return pltpu.make_async_copy(
                x_ref.at[:, pl.ds(b * BLK, BLK)], slab.at[p], in_sems.at[p])

        in_copy(0, 0).start()
        ones_r = jnp.ones((B, 1, BLK * WS), jnp.bfloat16)
        dnum = (((2,), (1,)), ((0,), (0,)))

        def stepA(b, carry):
            s, q = carry
            p = lax.rem(b, 2)

            @pl.when(b + 1 < NBLK)
            def _():
                in_copy(b + 1, lax.rem(b + 1, 2)).start()

            in_copy(b, p).wait()
            vf = slab[p]
            hpad[:, pl.ds(1 + b * BLK, BLK), 1:1 + WS, :] = (
                vf.astype(jnp.bfloat16))
            s = s + jnp.sum(vf, axis=(1, 2))
            q = q + jnp.sum(vf * vf, axis=(1, 2))
            return s, q

        s_acc, q_acc = lax.fori_loop(
            0, NBLK, stepA,
            (jnp.zeros((B, C), jnp.float32), jnp.zeros((B, C), jnp.float32)))

        stats_tx[0:B, :] = s_acc
        stats_tx[B:2 * B, :] = q_acc
        ex0 = pltpu.make_async_remote_copy(
            src_ref=stats_tx, dst_ref=stats_rx.at[0],
            send_sem=st_ssem.at[0], recv_sem=st_rsem.at[0],
            device_id=(1 - mx, my), device_id_type=pl.DeviceIdType.MESH)
        ex0.start()
        ex0.wait()
        comb = stats_tx[...] + stats_rx[0]
        stats_tx[...] = comb
        ex1 = pltpu.make_async_remote_copy(
            src_ref=stats_tx, dst_ref=stats_rx.at[1],
            send_sem=st_ssem.at[1], recv_sem=st_rsem.at[1],
            device_id=(mx, 1 - my), device_id_type=pl.DeviceIdType.MESH)
        ex1.start()
        ex1.wait()
        tot = comb + stats_rx[1]

        mean = tot[0:B, :] / N_GLOBAL
        var = tot[B:2 * B, :] / N_GLOBAL - mean * mean
        rstd = 1.0 / jnp.sqrt(var + EPS)

        rdma_row.wait()
        rdma_col.wait()
        rdma_cor.wait()
        hr = halo_row[...].astype(jnp.bfloat16)
        hc = halo_col[...].astype(jnp.bfloat16)
        hco = halo_cor[...].astype(jnp.bfloat16)

        row_top = jnp.where(mx == 1, hr, hpad[:, 1, 1:1 + WS, :])
        row_bot = jnp.where(mx == 0, hr, hpad[:, HS, 1:1 + WS, :])
        hpad[:, 0, 1:1 + WS, :] = row_top
        hpad[:, HS + 1, 1:1 + WS, :] = row_bot
        col_lft = jnp.where(my == 1, hc, hpad[:, 1:1 + HS, 1, :])
        col_rgt = jnp.where(my == 0, hc, hpad[:, 1:1 + HS, WS, :])
        hpad[:, 1:1 + HS, 0, :] = col_lft
        hpad[:, 1:1 + HS, WS + 1, :] = col_rgt

        def corner(r_remote, c_remote, hr_end, hc_end, local):
            return jnp.where(
                r_remote,
                jnp.where(c_remote, hco, hr_end),
                jnp.where(c_remote, hc_end, local))

        hpad[:, 0, 0, :] = corner(
            mx == 1, my == 1, hr[:, 0, :], hc[:, 0, :], hpad[:, 1, 1, :])
        hpad[:, 0, WS + 1, :] = corner(
            mx == 1, my == 0, hr[:, WS - 1, :], hc[:, 0, :],
            hpad[:, 1, WS, :])
        hpad[:, HS + 1, 0, :] = corner(
            mx == 0, my == 1, hr[:, 0, :], hc[:, HS - 1, :],
            hpad[:, HS, 1, :])
        hpad[:, HS + 1, WS + 1, :] = corner(
            mx == 0, my == 0, hr[:, WS - 1, :], hc[:, HS - 1, :],
            hpad[:, HS, WS, :])

        kv = k_ref[...].astype(jnp.bfloat16)
        wv = wp_ref[...].astype(jnp.bfloat16)
        ksum = jnp.sum(k_ref[...], axis=(0, 1))
        kkm4 = (mean * ksum).reshape(B, 1, 1, C)
        rstd4 = rstd.reshape(B, 1, 1, C)

        def out_copy(b, p):
            return pltpu.make_async_copy(
                outb.at[p], out_ref.at[:, pl.ds(b * BLK, BLK)],
                out_sems.at[p])

        def stepC(b, _):
            r0 = b * BLK
            p = lax.rem(b, 2)

            @pl.when(b >= 2)
            def _():
                out_copy(b - 2, p).wait()

            abl = os.environ.get("ABL", "")
            if abl == "nophaseC":
                res = hpad[:, pl.ds(1 + r0, BLK), 1:1 + WS, :].astype(
                    jnp.float32)
            else:
                kv32 = k_ref[...]
                acc32 = jnp.zeros((B, BLK, WS, C), jnp.float32)
                for dj in range(3):
                    wdj = hpad[:, pl.ds(r0, BLK + 2), dj:dj + WS, :
                               ].astype(jnp.float32)
                    for di in range(3):
                        acc32 = acc32 + (wdj[:, di:di + BLK, :, :]
                                         * kv32[di, dj, :])
                cf = (acc32 - kkm4) * rstd4
                if abl == "nosilu":
                    a = cf.astype(jnp.bfloat16)
                else:
                    a = (cf / (1.0 + jnp.exp(-cf))).astype(jnp.bfloat16)
                if abl == "nomm":
                    proj = a.astype(jnp.float32)
                else:
                    proj = jnp.dot(a.reshape(B * BLK * WS, C), wv,
                                   preferred_element_type=jnp.float32
                                   ).reshape(B, BLK, WS, C)
                res = (hpad[:, pl.ds(1 + r0, BLK), 1:1 + WS, :]
                       .astype(jnp.float32) + proj)
            outb[p] = res.astype(jnp.bfloat16)
            out_copy(b, p).start()
            return 0

        lax.fori_loop(0, NBLK, stepC, 0)
        out_copy(NBLK - 2, 0).wait()
        out_copy(NBLK - 1, 1).wait()

    out_shape = jax.ShapeDtypeStruct((B, HS, WS, C), jnp.bfloat16)
    return pl.pallas_call(
        body,
        out_shape=out_shape,
        in_specs=[
            pl.BlockSpec(memory_space=pl.ANY),
            pl.BlockSpec(memory_space=pltpu.MemorySpace.VMEM),
            pl.BlockSpec(memory_space=pltpu.MemorySpace.VMEM),
            pl.BlockSpec(memory_space=pltpu.MemorySpace.VMEM),
            pl.BlockSpec(memory_space=pltpu.MemorySpace.VMEM),
            pl.BlockSpec(memory_space=pltpu.MemorySpace.VMEM),
        ],
        out_specs=pl.BlockSpec(memory_space=pl.ANY),
        scratch_shapes=[
            pltpu.VMEM((B, HS + 2, WS + 2, C), jnp.bfloat16),
            pltpu.VMEM((2, B, BLK, WS, C), jnp.float32),
            pltpu.VMEM((2, B, BLK, WS, C), jnp.bfloat16),
            pltpu.VMEM((B, WS, C), jnp.float32),
            pltpu.VMEM((B, HS, C), jnp.float32),
            pltpu.VMEM((B, C), jnp.float32),
            pltpu.VMEM((2 * B, C), jnp.float32),
            pltpu.VMEM((2, 2 * B, C), jnp.float32),
            pltpu.SemaphoreType.DMA((2,)),
            pltpu.SemaphoreType.DMA((2,)),
            pltpu.SemaphoreType.DMA((3,)),
            pltpu.SemaphoreType.DMA((3,)),
            pltpu.SemaphoreType.DMA((2,)),
            pltpu.SemaphoreType.DMA((2,)),
        ],
        compiler_params=pltpu.CompilerParams(
            vmem_limit_bytes=128 * 1024 * 1024,
        ),
    )(x, k, Wp, row_send, col_send, corner_send)


# device time: 188575 ns/iter; 1.5357x vs baseline; 1.0001x over previous
import jax
import jax.numpy as jnp
from jax import lax
from jax.experimental import pallas as pl
from jax.experimental.pallas import tpu as pltpu

B, HS, WS, C = 2, 256, 256, 128
N_GLOBAL = 512 * 512
EPS = 1e-5
BLK = 32
NBLK = HS // BLK


def kernel(x, k, Wp):
    mx = lax.axis_index("x")
    my = lax.axis_index("y")

    r_edge = jnp.where(mx == 0, HS - 1, 0)
    c_edge = jnp.where(my == 0, WS - 1, 0)
    row_send = lax.dynamic_slice_in_dim(x, r_edge, 1, axis=1).reshape(B, WS, C)
    col_send = lax.dynamic_slice_in_dim(x, c_edge, 1, axis=2).reshape(B, HS, C)
    corner_send = lax.dynamic_slice(
        x, (0, r_edge, c_edge, 0), (B, 1, 1, C)
    ).reshape(B, C)

    def body(x_ref, k_ref, wp_ref, row_s_ref, col_s_ref, cor_s_ref,
             out_ref,
             hpad, slab, outb,
             halo_row, halo_col, halo_cor,
             stats_tx, stats_rx,
             in_sems, out_sems, halo_ssem, halo_rsem, st_ssem, st_rsem):
        mx = lax.axis_index("x")
        my = lax.axis_index("y")

        rdma_row = pltpu.make_async_remote_copy(
            src_ref=row_s_ref, dst_ref=halo_row,
            send_sem=halo_ssem.at[0], recv_sem=halo_rsem.at[0],
            device_id=(1 - mx, my), device_id_type=pl.DeviceIdType.MESH)
        rdma_col = pltpu.make_async_remote_copy(
            src_ref=col_s_ref, dst_ref=halo_col,
            send_sem=halo_ssem.at[1], recv_sem=halo_rsem.at[1],
            device_id=(mx, 1 - my), device_id_type=pl.DeviceIdType.MESH)
        rdma_cor = pltpu.make_async_remote_copy(
            src_ref=cor_s_ref, dst_ref=halo_cor,
            send_sem=halo_ssem.at[2], recv_sem=halo_rsem.at[2],
            device_id=(1 - mx, 1 - my), device_id_type=pl.DeviceIdType.MESH)
        rdma_row.start()
        rdma_col.start()
        rdma_cor.start()

        def in_copy(b, p):
            return pltpu.make_async_copy(
                x_ref.at[:, pl.ds(b * BLK, BLK)], slab.at[p], in_sems.at[p])

        in_copy(0, 0).start()

        def stepA(b, carry):
            s, q = carry
            p = lax.rem(b, 2)

            @pl.when(b + 1 < NBLK)
            def _():
                in_copy(b + 1, lax.rem(b + 1, 2)).start()

            in_copy(b, p).wait()
            vf = slab[p]
            hpad[:, pl.ds(1 + b * BLK, BLK), 1:1 + WS, :] = (
                vf.astype(jnp.bfloat16))
            s = s + jnp.sum(vf, axis=(1, 2))
            q = q + jnp.sum(vf * vf, axis=(1, 2))
            return s, q

        s_acc, q_acc = lax.fori_loop(
            0, NBLK, stepA,
            (jnp.zeros((B, C), jnp.float32), jnp.zeros((B, C), jnp.float32)))

        stats_tx[0:B, :] = s_acc
        stats_tx[B:2 * B, :] = q_acc
        ex0 = pltpu.make_async_remote_copy(
            src_ref=stats_tx, dst_ref=stats_rx.at[0],
            send_sem=st_ssem.at[0], recv_sem=st_rsem.at[0],
            device_id=(1 - mx, my), device_id_type=pl.DeviceIdType.MESH)
        ex0.start()
        ex0.wait()
        comb = stats_tx[...] + stats_rx[0]
        stats_tx[...] = comb
        ex1 = pltpu.make_async_remote_copy(
            src_ref=stats_tx, dst_ref=stats_rx.at[1],
            send_sem=st_ssem.at[1], recv_sem=st_rsem.at[1],
            device_id=(mx, 1 - my), device_id_type=pl.DeviceIdType.MESH)
        ex1.start()
        ex1.wait()
        tot = comb + stats_rx[1]

        mean = tot[0:B, :] / N_GLOBAL
        var = tot[B:2 * B, :] / N_GLOBAL - mean * mean
        rstd = 1.0 / jnp.sqrt(var + EPS)

        rdma_row.wait()
        rdma_col.wait()
        rdma_cor.wait()
        hr = halo_row[...].astype(jnp.bfloat16)
        hc = halo_col[...].astype(jnp.bfloat16)
        hco = halo_cor[...].astype(jnp.bfloat16)

        row_top = jnp.where(mx == 1, hr, hpad[:, 1, 1:1 + WS, :])
        row_bot = jnp.where(mx == 0, hr, hpad[:, HS, 1:1 + WS, :])
        hpad[:, 0, 1:1 + WS, :] = row_top
        hpad[:, HS + 1, 1:1 + WS, :] = row_bot
        col_lft = jnp.where(my == 1, hc, hpad[:, 1:1 + HS, 1, :])
        col_rgt = jnp.where(my == 0, hc, hpad[:, 1:1 + HS, WS, :])
        hpad[:, 1:1 + HS, 0, :] = col_lft
        hpad[:, 1:1 + HS, WS + 1, :] = col_rgt

        def corner(r_remote, c_remote, hr_end, hc_end, local):
            return jnp.where(
                r_remote,
                jnp.where(c_remote, hco, hr_end),
                jnp.where(c_remote, hc_end, local))

        hpad[:, 0, 0, :] = corner(
            mx == 1, my == 1, hr[:, 0, :], hc[:, 0, :], hpad[:, 1, 1, :])
        hpad[:, 0, WS + 1, :] = corner(
            mx == 1, my == 0, hr[:, WS - 1, :], hc[:, 0, :],
            hpad[:, 1, WS, :])
        hpad[:, HS + 1, 0, :] = corner(
            mx == 0, my == 1, hr[:, 0, :], hc[:, HS - 1, :],
            hpad[:, HS, 1, :])
        hpad[:, HS + 1, WS + 1, :] = corner(
            mx == 0, my == 0, hr[:, WS - 1, :], hc[:, HS - 1, :],
            hpad[:, HS, WS, :])

        wv = wp_ref[...].astype(jnp.bfloat16)
        ksum = jnp.sum(k_ref[...], axis=(0, 1))
        kkm4 = (mean * ksum).reshape(B, 1, 1, C)
        rstd4 = rstd.reshape(B, 1, 1, C)

        def out_copy(b, p):
            return pltpu.make_async_copy(
                outb.at[p], out_ref.at[:, pl.ds(b * BLK, BLK)],
                out_sems.at[p])

        def stepC(b, _):
            r0 = b * BLK
            p = lax.rem(b, 2)

            @pl.when(b >= 2)
            def _():
                out_copy(b - 2, p).wait()

            kv32 = k_ref[...]
            acc32 = jnp.zeros((B, BLK, WS, C), jnp.float32)
            for dj in range(3):
                wdj = hpad[:, pl.ds(r0, BLK + 2), dj:dj + WS, :
                           ].astype(jnp.float32)
                for di in range(3):
                    acc32 = acc32 + (wdj[:, di:di + BLK, :, :]
                                     * kv32[di, dj, :])
            cf = (acc32 - kkm4) * rstd4
            a = (cf / (1.0 + jnp.exp(-cf))).astype(jnp.bfloat16)
            proj = jnp.dot(a.reshape(B * BLK * WS, C), wv,
                           preferred_element_type=jnp.float32
                           ).reshape(B, BLK, WS, C)
            res = (hpad[:, pl.ds(1 + r0, BLK), 1:1 + WS, :]
                   .astype(jnp.float32) + proj)
            outb[p] = res.astype(jnp.bfloat16)
            out_copy(b, p).start()
            return 0

        lax.fori_loop(0, NBLK, stepC, 0)
        out_copy(NBLK - 2, 0).wait()
        out_copy(NBLK - 1, 1).wait()

    out_shape = jax.ShapeDtypeStruct((B, HS, WS, C), jnp.bfloat16)
    return pl.pallas_call(
        body,
        out_shape=out_shape,
        in_specs=[
            pl.BlockSpec(memory_space=pl.ANY),
            pl.BlockSpec(memory_space=pltpu.MemorySpace.VMEM),
            pl.BlockSpec(memory_space=pltpu.MemorySpace.VMEM),
            pl.BlockSpec(memory_space=pltpu.MemorySpace.VMEM),
            pl.BlockSpec(memory_space=pltpu.MemorySpace.VMEM),
            pl.BlockSpec(memory_space=pltpu.MemorySpace.VMEM),
        ],
        out_specs=pl.BlockSpec(memory_space=pl.ANY),
        scratch_shapes=[
            pltpu.VMEM((B, HS + 2, WS + 2, C), jnp.bfloat16),
            pltpu.VMEM((2, B, BLK, WS, C), jnp.float32),
            pltpu.VMEM((2, B, BLK, WS, C), jnp.bfloat16),
            pltpu.VMEM((B, WS, C), jnp.float32),
            pltpu.VMEM((B, HS, C), jnp.float32),
            pltpu.VMEM((B, C), jnp.float32),
            pltpu.VMEM((2 * B, C), jnp.float32),
            pltpu.VMEM((2, 2 * B, C), jnp.float32),
            pltpu.SemaphoreType.DMA((2,)),
            pltpu.SemaphoreType.DMA((2,)),
            pltpu.SemaphoreType.DMA((3,)),
            pltpu.SemaphoreType.DMA((3,)),
            pltpu.SemaphoreType.DMA((2,)),
            pltpu.SemaphoreType.DMA((2,)),
        ],
        compiler_params=pltpu.CompilerParams(
            vmem_limit_bytes=128 * 1024 * 1024,
        ),
    )(x, k, Wp, row_send, col_send, corner_send)
